# initial kernel scaffold (unmeasured)
import functools

import jax
import jax.numpy as jnp
from jax import lax
from jax.experimental import pallas as pl
from jax.experimental.pallas import tpu as pltpu

N_DEV = 4
N_TOK = 2048
D_MODEL = 1024
E_LOCAL = 8
N_EXPERTS = 32
CHUNK = N_TOK // N_DEV


def kernel(x, router_W, route_idx, expert_W):
    def body(x_ref, rw_ref, idx_ref, ew_ref, out_ref,
             accbf_ref, rs_recv, ag_recv, send_sems, recv_sems):
        my_pos = lax.axis_index("i")
        left = lax.rem(my_pos - 1 + N_DEV, N_DEV)
        right = lax.rem(my_pos + 1, N_DEV)

        barrier_sem = pltpu.get_barrier_semaphore()
        for nbr in (left, right):
            pl.semaphore_signal(
                barrier_sem, inc=1,
                device_id=(nbr,), device_id_type=pl.DeviceIdType.MESH,
            )
        pl.semaphore_wait(barrier_sem, 2)

        xf = x_ref[:, :]
        scores = jnp.dot(xf, rw_ref[:, :],
                         preferred_element_type=jnp.float32)
        smax = jnp.max(scores, axis=-1, keepdims=True)
        p = jnp.exp(scores - smax)
        probs = p / jnp.sum(p, axis=-1, keepdims=True)

        idx0 = idx_ref[:, 0:1]
        idx1 = idx_ref[:, 1:2]
        eids = lax.broadcasted_iota(jnp.int32, (1, N_EXPERTS), 1)
        g0 = jnp.sum(jnp.where(idx0 == eids, probs, 0.0), axis=-1,
                     keepdims=True)
        g1 = jnp.sum(jnp.where(idx1 == eids, probs, 0.0), axis=-1,
                     keepdims=True)
        gsum = g0 + g1
        g0n = g0 / gsum
        g1n = g1 / gsum

        local_ids = my_pos * E_LOCAL + lax.broadcasted_iota(
            jnp.int32, (1, E_LOCAL), 1)
        w = (jnp.where(idx0 == local_ids, g0n, 0.0)
             + jnp.where(idx1 == local_ids, g1n, 0.0))

        x_bf = xf.astype(jnp.bfloat16)
        acc = jnp.zeros((N_TOK, D_MODEL), jnp.float32)
        for e in range(E_LOCAL):
            we = w[:, e:e + 1].astype(jnp.bfloat16)
            xm = x_bf * we
            acc = acc + jnp.dot(xm, ew_ref[e].astype(jnp.bfloat16),
                                preferred_element_type=jnp.float32)

        for c in range(N_DEV):
            accbf_ref[c] = acc[c * CHUNK:(c + 1) * CHUNK, :].astype(
                jnp.bfloat16)

        for s in range(N_DEV - 1):
            send_c = lax.rem(my_pos - s + N_DEV, N_DEV)
            rdma = pltpu.make_async_remote_copy(
                src_ref=accbf_ref.at[send_c],
                dst_ref=rs_recv.at[s],
                send_sem=send_sems.at[s],
                recv_sem=recv_sems.at[s],
                device_id=(right,),
                device_id_type=pl.DeviceIdType.MESH,
            )
            rdma.start()
            rdma.wait()
            recv_c = lax.rem(my_pos - s - 1 + N_DEV, N_DEV)
            accbf_ref[recv_c] = (
                accbf_ref[recv_c].astype(jnp.float32)
                + rs_recv[s].astype(jnp.float32)
            ).astype(jnp.bfloat16)

        own_c = lax.rem(my_pos + 1, N_DEV)
        out_ref[pl.ds(own_c * CHUNK, CHUNK), :] = accbf_ref[own_c].astype(
            jnp.float32)

        for t in range(N_DEV - 1):
            send_c = lax.rem(my_pos + 1 - t + N_DEV, N_DEV)
            rdma = pltpu.make_async_remote_copy(
                src_ref=accbf_ref.at[send_c],
                dst_ref=ag_recv.at[t],
                send_sem=send_sems.at[3 + t],
                recv_sem=recv_sems.at[3 + t],
                device_id=(right,),
                device_id_type=pl.DeviceIdType.MESH,
            )
            rdma.start()
            rdma.wait()
            recv_c = lax.rem(my_pos - t + N_DEV, N_DEV)
            accbf_ref[recv_c] = ag_recv[t]
            out_ref[pl.ds(recv_c * CHUNK, CHUNK), :] = ag_recv[t].astype(
                jnp.float32)

    return pl.pallas_call(
        body,
        out_shape=jax.ShapeDtypeStruct((N_TOK, D_MODEL), jnp.float32),
        in_specs=[
            pl.BlockSpec(memory_space=pltpu.VMEM),
            pl.BlockSpec(memory_space=pltpu.VMEM),
            pl.BlockSpec(memory_space=pltpu.VMEM),
            pl.BlockSpec(memory_space=pltpu.VMEM),
        ],
        out_specs=pl.BlockSpec(memory_space=pltpu.VMEM),
        scratch_shapes=[
            pltpu.VMEM((N_DEV, CHUNK, D_MODEL), jnp.bfloat16),
            pltpu.VMEM((N_DEV - 1, CHUNK, D_MODEL), jnp.bfloat16),
            pltpu.VMEM((N_DEV - 1, CHUNK, D_MODEL), jnp.bfloat16),
            pltpu.SemaphoreType.DMA((2 * (N_DEV - 1),)),
            pltpu.SemaphoreType.DMA((2 * (N_DEV - 1),)),
        ],
        compiler_params=pltpu.CompilerParams(collective_id=0),
    )(x, router_W, route_idx, expert_W)


# baseline (device time: 144489 ns/iter reference)
import functools

import jax
import jax.numpy as jnp
from jax import lax
from jax.experimental import pallas as pl
from jax.experimental.pallas import tpu as pltpu

N_DEV = 4
N_TOK = 2048
D_MODEL = 1024
E_LOCAL = 8
N_EXPERTS = 32
CHUNK = N_TOK // N_DEV


def kernel(x, router_W, route_idx, expert_W):
    def body(x_ref, rw_ref, idx_ref, ew_hbm, out_ref,
             ew_vmem, ewbf_ref, xbf_ref, acc_ref, accbf_ref,
             rs_recv, ag_recv, copy_sems, send_sems, recv_sems):
        my_pos = lax.axis_index("i")
        left = lax.rem(my_pos - 1 + N_DEV, N_DEV)
        right = lax.rem(my_pos + 1, N_DEV)

        barrier_sem = pltpu.get_barrier_semaphore()
        for nbr in (left, right):
            pl.semaphore_signal(
                barrier_sem, inc=1,
                device_id=(nbr,), device_id_type=pl.DeviceIdType.MESH,
            )
        pl.semaphore_wait(barrier_sem, 2)

        scores = jnp.dot(x_ref[:, :], rw_ref[:, :],
                         preferred_element_type=jnp.float32)
        smax = jnp.max(scores, axis=-1, keepdims=True)
        p = jnp.exp(scores - smax)
        probs = p / jnp.sum(p, axis=-1, keepdims=True)

        idx0 = idx_ref[:, 0:1]
        idx1 = idx_ref[:, 1:2]
        eids = lax.broadcasted_iota(jnp.int32, (1, N_EXPERTS), 1)
        g0 = jnp.sum(jnp.where(idx0 == eids, probs, 0.0), axis=-1,
                     keepdims=True)
        g1 = jnp.sum(jnp.where(idx1 == eids, probs, 0.0), axis=-1,
                     keepdims=True)
        gsum = g0 + g1
        g0n = g0 / gsum
        g1n = g1 / gsum

        local_ids = my_pos * E_LOCAL + lax.broadcasted_iota(
            jnp.int32, (1, E_LOCAL), 1)
        w = (jnp.where(idx0 == local_ids, g0n, 0.0)
             + jnp.where(idx1 == local_ids, g1n, 0.0))

        def ew_copy(e, slot):
            return pltpu.make_async_copy(
                ew_hbm.at[e], ew_vmem.at[slot], copy_sems.at[slot])

        ew_copy(0, 0).start()
        for b in range(N_DEV):
            xbf_ref[pl.ds(b * CHUNK, CHUNK), :] = x_ref[
                pl.ds(b * CHUNK, CHUNK), :].astype(jnp.bfloat16)

        for e in range(E_LOCAL):
            slot = e % 2
            if e + 1 < E_LOCAL:
                ew_copy(e + 1, (e + 1) % 2).start()
            ew_copy(e, slot).wait()
            ewbf_ref[:, :] = ew_vmem[slot].astype(jnp.bfloat16)
            for b in range(N_DEV):
                rows = pl.ds(b * CHUNK, CHUNK)
                part = jnp.dot(xbf_ref[rows, :], ewbf_ref[:, :],
                               preferred_element_type=jnp.float32)
                gated = w[b * CHUNK:(b + 1) * CHUNK, e:e + 1] * part
                if e == 0:
                    acc_ref[rows, :] = gated
                else:
                    acc_ref[rows, :] = acc_ref[rows, :] + gated

        for c in range(N_DEV):
            accbf_ref[c] = acc_ref[c * CHUNK:(c + 1) * CHUNK, :].astype(
                jnp.bfloat16)

        for s in range(N_DEV - 1):
            send_c = lax.rem(my_pos - s + N_DEV, N_DEV)
            rdma = pltpu.make_async_remote_copy(
                src_ref=accbf_ref.at[send_c],
                dst_ref=rs_recv.at[s],
                send_sem=send_sems.at[s],
                recv_sem=recv_sems.at[s],
                device_id=(right,),
                device_id_type=pl.DeviceIdType.MESH,
            )
            rdma.start()
            rdma.wait()
            recv_c = lax.rem(my_pos - s - 1 + N_DEV, N_DEV)
            accbf_ref[recv_c] = (
                accbf_ref[recv_c].astype(jnp.float32)
                + rs_recv[s].astype(jnp.float32)
            ).astype(jnp.bfloat16)

        own_c = lax.rem(my_pos + 1, N_DEV)
        out_ref[pl.ds(own_c * CHUNK, CHUNK), :] = accbf_ref[own_c].astype(
            jnp.float32)

        for t in range(N_DEV - 1):
            send_c = lax.rem(my_pos + 1 - t + N_DEV, N_DEV)
            rdma = pltpu.make_async_remote_copy(
                src_ref=accbf_ref.at[send_c],
                dst_ref=ag_recv.at[t],
                send_sem=send_sems.at[3 + t],
                recv_sem=recv_sems.at[3 + t],
                device_id=(right,),
                device_id_type=pl.DeviceIdType.MESH,
            )
            rdma.start()
            rdma.wait()
            recv_c = lax.rem(my_pos - t + N_DEV, N_DEV)
            accbf_ref[recv_c] = ag_recv[t]
            out_ref[pl.ds(recv_c * CHUNK, CHUNK), :] = ag_recv[t].astype(
                jnp.float32)

    return pl.pallas_call(
        body,
        out_shape=jax.ShapeDtypeStruct((N_TOK, D_MODEL), jnp.float32),
        in_specs=[
            pl.BlockSpec(memory_space=pltpu.VMEM),
            pl.BlockSpec(memory_space=pltpu.VMEM),
            pl.BlockSpec(memory_space=pltpu.VMEM),
            pl.BlockSpec(memory_space=pl.ANY),
        ],
        out_specs=pl.BlockSpec(memory_space=pltpu.VMEM),
        scratch_shapes=[
            pltpu.VMEM((2, D_MODEL, D_MODEL), jnp.float32),
            pltpu.VMEM((D_MODEL, D_MODEL), jnp.bfloat16),
            pltpu.VMEM((N_TOK, D_MODEL), jnp.bfloat16),
            pltpu.VMEM((N_TOK, D_MODEL), jnp.float32),
            pltpu.VMEM((N_DEV, CHUNK, D_MODEL), jnp.bfloat16),
            pltpu.VMEM((N_DEV - 1, CHUNK, D_MODEL), jnp.bfloat16),
            pltpu.VMEM((N_DEV - 1, CHUNK, D_MODEL), jnp.bfloat16),
            pltpu.SemaphoreType.DMA((2,)),
            pltpu.SemaphoreType.DMA((2 * (N_DEV - 1),)),
            pltpu.SemaphoreType.DMA((2 * (N_DEV - 1),)),
        ],
        compiler_params=pltpu.CompilerParams(
            collective_id=0, vmem_limit_bytes=128 * 1024 * 1024),
    )(x, router_W, route_idx, expert_W)


# device time: 65483 ns/iter; 2.2065x vs baseline; 2.2065x over previous
import jax
import jax.numpy as jnp
from jax import lax
from jax.experimental import pallas as pl
from jax.experimental.pallas import tpu as pltpu

N_DEV = 4
N_TOK = 2048
D_MODEL = 1024
E_LOCAL = 8
N_EXPERTS = 32
CHUNK = N_TOK // N_DEV
HALF = D_MODEL // 2


def kernel(x, router_W, route_idx, expert_W):
    def body(x_ref, rw_ref, idx_ref, ew_hbm, out_ref,
             ew_vmem, ewbf_ref, xbf_ref, w_ref, accA, accB, rsA, rsB,
             copy_sems, sendA, recvA, sendB, recvB):
        p = lax.axis_index("i")
        left = lax.rem(p - 1 + N_DEV, N_DEV)
        right = lax.rem(p + 1, N_DEV)

        def cid(k):
            return lax.rem(p + k + 2 * N_DEV, N_DEV)

        barrier_sem = pltpu.get_barrier_semaphore()
        for nbr in (left, right):
            pl.semaphore_signal(
                barrier_sem, inc=1,
                device_id=(nbr,), device_id_type=pl.DeviceIdType.MESH,
            )
        pl.semaphore_wait(barrier_sem, 2)

        def ew_copy(e, slot):
            return pltpu.make_async_copy(
                ew_hbm.at[e], ew_vmem.at[slot], copy_sems.at[slot])

        ew_copy(0, 0).start()

        scores = jnp.dot(x_ref[:, :], rw_ref[:, :],
                         preferred_element_type=jnp.float32)
        smax = jnp.max(scores, axis=-1, keepdims=True)
        pexp = jnp.exp(scores - smax)
        probs = pexp / jnp.sum(pexp, axis=-1, keepdims=True)

        idx0 = idx_ref[:, 0:1]
        idx1 = idx_ref[:, 1:2]
        eids = lax.broadcasted_iota(jnp.int32, (1, N_EXPERTS), 1)
        g0 = jnp.sum(jnp.where(idx0 == eids, probs, 0.0), axis=-1,
                     keepdims=True)
        g1 = jnp.sum(jnp.where(idx1 == eids, probs, 0.0), axis=-1,
                     keepdims=True)
        gsum = g0 + g1

        local_ids = p * E_LOCAL + lax.broadcasted_iota(
            jnp.int32, (1, E_LOCAL), 1)
        w_ref[:, :] = (jnp.where(idx0 == local_ids, g0 / gsum, 0.0)
                       + jnp.where(idx1 == local_ids, g1 / gsum, 0.0))

        for b in range(N_DEV):
            xbf_ref[pl.ds(b * CHUNK, CHUNK), :] = x_ref[
                pl.ds(b * CHUNK, CHUNK), :].astype(jnp.bfloat16)

        sent = []

        def compute_block(bidx):
            rows = pl.ds(bidx * CHUNK, CHUNK)
            acc = None
            for e in range(E_LOCAL):
                part = jnp.dot(xbf_ref[rows, :], ewbf_ref[e],
                               preferred_element_type=jnp.float32)
                gated = w_ref[rows, e:e + 1] * part
                acc = gated if acc is None else acc + gated
            accA[bidx] = acc[:, :HALF].astype(jnp.bfloat16)
            accB[bidx] = acc[:, HALF:].astype(jnp.bfloat16)

        rows0 = pl.ds(cid(0) * CHUNK, CHUNK)
        acc0 = None
        for e in range(E_LOCAL):
            slot = e % 2
            if e + 1 < E_LOCAL:
                ew_copy(e + 1, (e + 1) % 2).start()
            ew_copy(e, slot).wait()
            ewbf_ref[e] = ew_vmem[slot].astype(jnp.bfloat16)
            part = jnp.dot(xbf_ref[rows0, :], ewbf_ref[e],
                           preferred_element_type=jnp.float32)
            gated = w_ref[rows0, e:e + 1] * part
            acc0 = gated if acc0 is None else acc0 + gated
        accA[cid(0)] = acc0[:, :HALF].astype(jnp.bfloat16)
        accB[cid(0)] = acc0[:, HALF:].astype(jnp.bfloat16)

        def rs_desc(ring, s):
            if ring == 0:
                return pltpu.make_async_remote_copy(
                    src_ref=accA.at[cid(-s)], dst_ref=rsA.at[s],
                    send_sem=sendA.at[s], recv_sem=recvA.at[s],
                    device_id=(right,), device_id_type=pl.DeviceIdType.MESH)
            return pltpu.make_async_remote_copy(
                src_ref=accB.at[cid(s)], dst_ref=rsB.at[s],
                send_sem=sendB.at[s], recv_sem=recvB.at[s],
                device_id=(left,), device_id_type=pl.DeviceIdType.MESH)

        def rs_add_a(s):
            c = cid(-s - 1)
            accA[c] = (accA[c].astype(jnp.float32)
                       + rsA[s].astype(jnp.float32)).astype(jnp.bfloat16)

        def rs_add_b(s):
            c = cid(s + 1)
            accB[c] = (accB[c].astype(jnp.float32)
                       + rsB[s].astype(jnp.float32)).astype(jnp.bfloat16)

        dA0 = rs_desc(0, 0); dA0.start(); sent.append(dA0)
        dB0 = rs_desc(1, 0); dB0.start(); sent.append(dB0)

        compute_block(cid(1))
        compute_block(cid(-1))

        dA0.wait_recv(); rs_add_a(0)
        dB0.wait_recv(); rs_add_b(0)
        dA1 = rs_desc(0, 1); dA1.start(); sent.append(dA1)
        dB1 = rs_desc(1, 1); dB1.start(); sent.append(dB1)

        compute_block(cid(2))

        dA1.wait_recv(); rs_add_a(1)
        dB1.wait_recv(); rs_add_b(1)
        dA2 = rs_desc(0, 2); dA2.start(); sent.append(dA2)
        dB2 = rs_desc(1, 2); dB2.start(); sent.append(dB2)

        dA2.wait_recv(); rs_add_a(2)
        dB2.wait_recv(); rs_add_b(2)

        out_ref[pl.ds(cid(1) * CHUNK, CHUNK), pl.ds(0, HALF)] = accA[cid(1)]
        out_ref[pl.ds(cid(-1) * CHUNK, CHUNK), pl.ds(HALF, HALF)] = (
            accB[cid(-1)])

        def out_half(c, ring):
            return out_ref.at[pl.ds(c * CHUNK, CHUNK),
                              pl.ds(0 if ring == 0 else HALF, HALF)]

        for t in range(N_DEV - 1):
            sa = out_half(cid(1 - t), 0)
            agA = pltpu.make_async_remote_copy(
                src_ref=sa, dst_ref=sa,
                send_sem=sendA.at[3 + t], recv_sem=recvA.at[3 + t],
                device_id=(right,), device_id_type=pl.DeviceIdType.MESH)
            sb = out_half(cid(t - 1), 1)
            agB = pltpu.make_async_remote_copy(
                src_ref=sb, dst_ref=sb,
                send_sem=sendB.at[3 + t], recv_sem=recvB.at[3 + t],
                device_id=(left,), device_id_type=pl.DeviceIdType.MESH)
            agA.start(); sent.append(agA)
            agB.start(); sent.append(agB)
            rxA = out_half(cid(-t), 0)
            pltpu.make_async_remote_copy(
                src_ref=rxA, dst_ref=rxA,
                send_sem=sendA.at[3 + t], recv_sem=recvA.at[3 + t],
                device_id=(right,), device_id_type=pl.DeviceIdType.MESH,
            ).wait_recv()
            rxB = out_half(cid(t), 1)
            pltpu.make_async_remote_copy(
                src_ref=rxB, dst_ref=rxB,
                send_sem=sendB.at[3 + t], recv_sem=recvB.at[3 + t],
                device_id=(left,), device_id_type=pl.DeviceIdType.MESH,
            ).wait_recv()

        for d in sent:
            d.wait_send()

    grid = ()
    return pl.pallas_call(
        body,
        out_shape=jax.ShapeDtypeStruct((N_TOK, D_MODEL), jnp.bfloat16),
        in_specs=[
            pl.BlockSpec(memory_space=pltpu.VMEM),
            pl.BlockSpec(memory_space=pltpu.VMEM),
            pl.BlockSpec(memory_space=pltpu.VMEM),
            pl.BlockSpec(memory_space=pl.ANY),
        ],
        out_specs=pl.BlockSpec(memory_space=pltpu.VMEM),
        scratch_shapes=[
            pltpu.VMEM((2, D_MODEL, D_MODEL), jnp.float32),
            pltpu.VMEM((E_LOCAL, D_MODEL, D_MODEL), jnp.bfloat16),
            pltpu.VMEM((N_TOK, D_MODEL), jnp.bfloat16),
            pltpu.VMEM((N_TOK, E_LOCAL), jnp.float32),
            pltpu.VMEM((N_DEV, CHUNK, HALF), jnp.bfloat16),
            pltpu.VMEM((N_DEV, CHUNK, HALF), jnp.bfloat16),
            pltpu.VMEM((N_DEV - 1, CHUNK, HALF), jnp.bfloat16),
            pltpu.VMEM((N_DEV - 1, CHUNK, HALF), jnp.bfloat16),
            pltpu.SemaphoreType.DMA((2,)),
            pltpu.SemaphoreType.DMA((2 * (N_DEV - 1),)),
            pltpu.SemaphoreType.DMA((2 * (N_DEV - 1),)),
            pltpu.SemaphoreType.DMA((2 * (N_DEV - 1),)),
            pltpu.SemaphoreType.DMA((2 * (N_DEV - 1),)),
        ],
        compiler_params=pltpu.CompilerParams(
            collective_id=0, vmem_limit_bytes=128 * 1024 * 1024),
    )(x, router_W, route_idx, expert_W)
